# baseline (speedup 1.0000x reference)
import functools

import numpy as np
import jax
import jax.numpy as jnp
from jax import lax
from jax.experimental import pallas as pl
from jax.experimental.pallas import tpu as pltpu
from jax.experimental.pallas import tpu_sc as plsc

E = 320000
N2 = 2 * E
NNODES = 10000
FP = 10240
NT = 16
CHUNK = N2 // NT
W = 10000
NWIN = CHUNK // W
EC = E // NT
NWE = EC // W
RBITS = 11
R = 1 << RBITS
NPASS = 6
E_BLOCK = 2000

_mesh = plsc.VectorSubcoreMesh(core_axis_name="c", subcore_axis_name="s",
                               num_cores=1)
_sc_params = pltpu.CompilerParams(needs_layout_passes=False)


def _i32(x):
    return np.int32(x)


def _al(x):
    return pl.multiple_of(x, 8)


def _ssp(x):
    return jnp.maximum(x, 0.0) + jnp.log1p(jnp.exp(-jnp.abs(x))) - 0.6931471805599453


def _mlp_body(feat_ref, w1_ref, b1_ref, w2_ref, b2_ref, w3_ref, b3_ref, em_ref):
    h = jnp.dot(feat_ref[...], w1_ref[...], preferred_element_type=jnp.float32)
    h = _ssp(h + b1_ref[...])
    h = jnp.dot(h, w2_ref[...], preferred_element_type=jnp.float32)
    h = _ssp(h + b2_ref[...])
    em = jnp.dot(h, w3_ref[...], preferred_element_type=jnp.float32) + b3_ref[...]
    em_ref[...] = em


def _mlp_pallas(feat, W1, b1, W2, b2, W3, b3):
    grid = (E // E_BLOCK,)
    return pl.pallas_call(
        _mlp_body,
        grid=grid,
        in_specs=[
            pl.BlockSpec((E_BLOCK, 128), lambda i: (i, np.int32(0))),
            pl.BlockSpec((128, 128), lambda i: (np.int32(0), np.int32(0))),
            pl.BlockSpec((1, 128), lambda i: (np.int32(0), np.int32(0))),
            pl.BlockSpec((128, 64), lambda i: (np.int32(0), np.int32(0))),
            pl.BlockSpec((1, 64), lambda i: (np.int32(0), np.int32(0))),
            pl.BlockSpec((64, 1), lambda i: (np.int32(0), np.int32(0))),
            pl.BlockSpec((1, 1), lambda i: (np.int32(0), np.int32(0))),
        ],
        out_specs=pl.BlockSpec((E_BLOCK, 1), lambda i: (i, np.int32(0))),
        out_shape=jax.ShapeDtypeStruct((E, 1), jnp.float32),
    )(feat, W1, b1.reshape(1, 128), W2, b2.reshape(1, 64), W3, b3.reshape(1, 1))


def _digit_of(klo, khi, p):
    s = RBITS * p
    if s >= 32:
        return jnp.bitwise_and(
            lax.shift_right_logical(khi, _i32(s - 32)), _i32(R - 1))
    d = lax.shift_right_logical(klo, _i32(s))
    if s + RBITS > 32:
        nhi = s + RBITS - 32
        hi = lax.shift_left(
            jnp.bitwise_and(khi, _i32((1 << nhi) - 1)), _i32(32 - s))
        d = jnp.bitwise_or(jnp.bitwise_and(d, _i32((1 << (32 - s)) - 1)), hi)
    return jnp.bitwise_and(d, _i32(R - 1))


def _sort_body(klo_hbm, khi_hbm, ord_a, ord_b,
               w_klo, w_khi, w_idx, w_pos,
               hist, tmp, acc, part, base,
               s_grid):
    t = lax.axis_index("s").astype(jnp.int32)
    cbase = t * _i32(CHUNK)
    iota = lax.iota(jnp.int32, 16)
    z16 = jnp.zeros((16,), jnp.int32)

    def zero_r(ref):
        def zb(i, off):
            ref[pl.ds(off, 16)] = z16
            return off + _i32(16)
        lax.fori_loop(0, R // 16, zb, _i32(0), unroll=4)

    def stage(wb, p):
        if p == 0:
            pltpu.sync_copy(klo_hbm.at[pl.ds(_al(wb), W)], w_klo)
            pltpu.sync_copy(khi_hbm.at[pl.ds(_al(wb), W)], w_khi)
        else:
            src = ord_a if p % 2 == 1 else ord_b
            pltpu.sync_copy(src.at[pl.ds(_al(wb), W)], w_idx)
            pltpu.sync_copy(klo_hbm.at[w_idx], w_klo)
            pltpu.sync_copy(khi_hbm.at[w_idx], w_khi)

    for p in range(NPASS):
        zero_r(hist)

        def b1_win(w, wb, p=p):
            stage(wb, p)

            def vb(i, off, p=p):
                klo = w_klo[pl.ds(off, 16)]
                khi = w_khi[pl.ds(off, 16)]
                d = _digit_of(klo, khi, p)
                occ, last = plsc.scan_count(d)
                plsc.addupdate_scatter(hist, [d], occ, mask=last)
                return off + _i32(16)
            lax.fori_loop(0, W // 16, vb, _i32(0), unroll=4)
            return wb + _i32(W)
        lax.fori_loop(0, NWIN, b1_win, cbase)

        pltpu.sync_copy(hist, s_grid.at[t])
        plsc.subcore_barrier()

        zero_r(acc)
        zero_r(part)
        for tp in range(NT):
            pltpu.sync_copy(s_grid.at[_i32(tp)], tmp)
            fvec = z16 + (_i32(tp) < t).astype(jnp.int32)

            def ab(i, off):
                sl = pl.ds(off, 16)
                v = tmp[sl]
                acc[sl] = acc[sl] + v
                part[sl] = part[sl] + v * fvec
                return off + _i32(16)
            lax.fori_loop(0, R // 16, ab, _i32(0), unroll=4)

        def bb(i, carry):
            off, tot = carry
            sl = pl.ds(off, 16)
            a = acc[sl]
            incl = plsc.cumsum(a)
            base[sl] = (incl - a) + (z16 + tot) + part[sl]
            return (off + _i32(16), tot + jnp.sum(a, dtype=jnp.int32))
        lax.fori_loop(0, R // 16, bb, (_i32(0), _i32(0)))

        def b3_win(w, wb, p=p):
            stage(wb, p)

            def vb(i, off, p=p, wb=wb):
                sl = pl.ds(off, 16)
                klo = w_klo[sl]
                khi = w_khi[sl]
                if p == 0:
                    w_idx[sl] = (iota + off) + wb
                d = _digit_of(klo, khi, p)
                occ, last = plsc.scan_count(d)
                b = plsc.load_gather(base, [d])
                w_pos[sl] = b + occ - _i32(1)
                plsc.addupdate_scatter(base, [d], occ, mask=last)
                return off + _i32(16)
            lax.fori_loop(0, W // 16, vb, _i32(0))
            dst = ord_a if p % 2 == 0 else ord_b
            pltpu.sync_copy(w_idx, dst.at[w_pos])
            return wb + _i32(W)
        lax.fori_loop(0, NWIN, b3_win, cbase)
        plsc.subcore_barrier()


def _sort_kernel(klo, khi):
    f = pl.kernel(
        _sort_body,
        out_type=(jax.ShapeDtypeStruct((N2,), jnp.int32),
                  jax.ShapeDtypeStruct((N2,), jnp.int32)),
        mesh=_mesh,
        compiler_params=_sc_params,
        scratch_types=[
            pltpu.VMEM((W,), jnp.int32),
            pltpu.VMEM((W,), jnp.int32),
            pltpu.VMEM((W,), jnp.int32),
            pltpu.VMEM((W,), jnp.int32),
            pltpu.VMEM((R,), jnp.int32),
            pltpu.VMEM((R,), jnp.int32),
            pltpu.VMEM((R,), jnp.int32),
            pltpu.VMEM((R,), jnp.int32),
            pltpu.VMEM((R,), jnp.int32),
            pltpu.VMEM_SHARED((NT, R), jnp.int32),
        ],
    )
    return f(klo, khi)[1]


def _pair_body(em_hbm, ec_hbm, ord_hbm, real2_hbm, ec2_hbm,
               ux_hbm, uy_hbm, uz_hbm,
               forces_hbm, emc_hbm,
               w_em, w_j, w_e, w_real, w_ec2,
               w_ux, w_uy, w_uz, w_fx, w_fy, w_fz,
               s_sums, s_cnt, s_forces):
    t = lax.axis_index("s").astype(jnp.int32)
    iota = lax.iota(jnp.int32, 16)
    onef = jnp.full((16,), 1.0, jnp.float32)
    zf16 = jnp.zeros((16,), jnp.float32)

    def of(i, off):
        w_fz[pl.ds(off, 16)] = onef
        w_fx[pl.ds(off, 16)] = zf16
        return off + _i32(16)
    lax.fori_loop(0, W // 16, of, _i32(0), unroll=4)

    nwords = FP * 3 // NT
    rb = t * _i32(nwords)
    pltpu.sync_copy(w_fx.at[pl.ds(0, nwords)], s_forces.at[pl.ds(_al(rb), nwords)])
    nsum = FP // NT
    sb = t * _i32(nsum)
    pltpu.sync_copy(w_fx.at[pl.ds(0, nsum)], s_sums.at[pl.ds(_al(sb), nsum)])
    pltpu.sync_copy(w_fx.at[pl.ds(0, nsum)], s_cnt.at[pl.ds(_al(sb), nsum)])
    plsc.subcore_barrier()

    ebase = t * _i32(EC)

    def a1(w, wb):
        pltpu.sync_copy(em_hbm.at[pl.ds(_al(wb), W)], w_em)
        pltpu.sync_copy(ec_hbm.at[pl.ds(_al(wb), W)], w_j)
        pltpu.sync_copy(w_em, s_sums.at[w_j], add=True)
        pltpu.sync_copy(w_fz, s_cnt.at[w_j], add=True)
        return wb + _i32(W)
    lax.fori_loop(0, NWE, a1, ebase)
    plsc.subcore_barrier()

    def a2(w, wb):
        pltpu.sync_copy(em_hbm.at[pl.ds(_al(wb), W)], w_em)
        pltpu.sync_copy(ec_hbm.at[pl.ds(_al(wb), W)], w_j)
        pltpu.sync_copy(s_sums.at[w_j], w_ux)
        pltpu.sync_copy(s_cnt.at[w_j], w_uy)

        def vb(i, off):
            sl = pl.ds(off, 16)
            w_em[sl] = w_em[sl] - w_ux[sl] / jnp.maximum(w_uy[sl], 1.0)
            return off + _i32(16)
        lax.fori_loop(0, W // 16, vb, _i32(0), unroll=4)
        pltpu.sync_copy(w_em, emc_hbm.at[pl.ds(_al(wb), W)])
        return wb + _i32(W)
    lax.fori_loop(0, NWE, a2, ebase)
    plsc.subcore_barrier()

    def c_win(w, wb):
        pltpu.sync_copy(ord_hbm.at[pl.ds(_al(wb), W)], w_j)

        def v1(i, off):
            sl = pl.ds(off, 16)
            j = w_j[sl]
            ge = (j >= _i32(E)).astype(jnp.int32)
            w_e[sl] = j - ge * _i32(E)
            return off + _i32(16)
        lax.fori_loop(0, W // 16, v1, _i32(0), unroll=4)

        pltpu.sync_copy(emc_hbm.at[w_e], w_em)
        pltpu.sync_copy(real2_hbm.at[w_j], w_real)
        pltpu.sync_copy(ec2_hbm.at[w_j], w_ec2)
        pltpu.sync_copy(ux_hbm.at[w_j], w_ux)
        pltpu.sync_copy(uy_hbm.at[w_j], w_uy)
        pltpu.sync_copy(uz_hbm.at[w_j], w_uz)

        def v3(i, off):
            sl = pl.ds(off, 16)
            rows = iota + off
            prow = jnp.bitwise_xor(rows, _i32(1))
            rl = w_real[sl]
            prl = plsc.load_gather(w_real, [prow])
            emv = w_em[sl] * rl
            pem = plsc.load_gather(w_em, [prow]) * prl
            valid = (rl * prl) > 0.5
            ne = jnp.where(valid, (emv + pem) * 0.5, emv)
            e3 = w_ec2[sl] * _i32(3)
            w_fx[sl] = ne * w_ux[sl]
            w_fy[sl] = ne * w_uy[sl]
            w_fz[sl] = ne * w_uz[sl]
            w_e[sl] = e3
            w_j[sl] = e3 + _i32(1)
            w_ec2[sl] = e3 + _i32(2)
            return off + _i32(16)
        lax.fori_loop(0, W // 16, v3, _i32(0), unroll=4)

        pltpu.sync_copy(w_fx, s_forces.at[w_e], add=True)
        pltpu.sync_copy(w_fy, s_forces.at[w_j], add=True)
        pltpu.sync_copy(w_fz, s_forces.at[w_ec2], add=True)

        return wb + _i32(W)
    lax.fori_loop(0, NWIN, c_win, t * _i32(CHUNK))
    plsc.subcore_barrier()

    pltpu.sync_copy(s_forces.at[pl.ds(_al(rb), nwords)],
                    w_fx.at[pl.ds(0, nwords)])
    pltpu.sync_copy(w_fx.at[pl.ds(0, nwords)],
                    forces_hbm.at[pl.ds(_al(rb), nwords)])


def _pair_kernel(em, ec, rec, real2, ec2, ux, uy, uz):
    f = pl.kernel(
        _pair_body,
        out_type=(jax.ShapeDtypeStruct((FP * 3,), jnp.float32),
                  jax.ShapeDtypeStruct((E,), jnp.float32)),
        mesh=_mesh,
        compiler_params=_sc_params,
        scratch_types=[
            pltpu.VMEM((W,), jnp.float32),
            pltpu.VMEM((W,), jnp.int32),
            pltpu.VMEM((W,), jnp.int32),
            pltpu.VMEM((W,), jnp.float32),
            pltpu.VMEM((W,), jnp.int32),
            pltpu.VMEM((W,), jnp.float32),
            pltpu.VMEM((W,), jnp.float32),
            pltpu.VMEM((W,), jnp.float32),
            pltpu.VMEM((W,), jnp.float32),
            pltpu.VMEM((W,), jnp.float32),
            pltpu.VMEM((W,), jnp.float32),
            pltpu.VMEM_SHARED((FP,), jnp.float32),
            pltpu.VMEM_SHARED((FP,), jnp.float32),
            pltpu.VMEM_SHARED((FP * 3,), jnp.float32),
        ],
    )
    return f(em, ec, rec, real2, ec2, ux, uy, uz)


def kernel(features_for_direct_force, edge_vectors, edge_lengths, pos, edge_index,
           W1, b1, W2, b2, W3, b3):
    ec64 = edge_index[0]
    en64 = edge_index[1]
    ec = ec64.astype(jnp.int32)
    en = en64.astype(jnp.int32)
    unit_vec = edge_vectors / edge_lengths[:, None]

    em = _mlp_pallas(features_for_direct_force, W1, b1, W2, b2, W3, b3)[:, 0]

    gm = en64 > ec64[-1]
    gmf = gm.astype(jnp.float32)
    real2 = jnp.concatenate([jnp.ones((E,), jnp.float32), gmf])
    zero_e = jnp.zeros((E,), jnp.float32)
    ux = jnp.concatenate([unit_vec[:, 0], jnp.where(gm, -unit_vec[:, 0], zero_e)])
    uy = jnp.concatenate([unit_vec[:, 1], jnp.where(gm, -unit_vec[:, 1], zero_e)])
    uz = jnp.concatenate([unit_vec[:, 2], jnp.where(gm, -unit_vec[:, 2], zero_e)])
    ec2 = jnp.concatenate([ec, jnp.where(gm, en, jnp.int32(NNODES))])
    en2 = jnp.concatenate([en, jnp.where(gm, ec, jnp.int32(0))])
    el2 = jnp.concatenate([edge_lengths, jnp.where(gm, edge_lengths, zero_e)])
    s_abs = jnp.abs(unit_vec).sum(axis=1)
    uvs2 = jnp.concatenate([s_abs, jnp.where(gm, s_abs, zero_e)])

    key = (ec2.astype(jnp.int64) + en2.astype(jnp.int64)
           + (1e10 * el2).astype(jnp.int64)
           + (1e10 * uvs2).astype(jnp.int64))
    is_real = jnp.concatenate([jnp.ones((E,), bool), gm])
    key = jnp.where(is_real, key, jnp.iinfo(jnp.int64).max)
    klo = (key & 0xFFFFFFFF).astype(jnp.int32)
    khi = lax.shift_right_logical(key, np.int64(32)).astype(jnp.int32)

    rec = _sort_kernel(klo, khi)

    forces_fp, _emc = _pair_kernel(em, ec, rec, real2, ec2, ux, uy, uz)
    return forces_fp.reshape(FP, 3)[:NNODES]


# speedup vs baseline: 3.4204x; 3.4204x over previous
import functools

import numpy as np
import jax
import jax.numpy as jnp
from jax import lax
from jax.experimental import pallas as pl
from jax.experimental.pallas import tpu as pltpu
from jax.experimental.pallas import tpu_sc as plsc

E = 320000
N2 = 2 * E
NNODES = 10000
FP = 10240
NT = 16
CHUNK = N2 // NT
W = 10000
NWIN = CHUNK // W
EC = E // NT
NWE = EC // W
RBITS = 11
R = 1 << RBITS
NPASS = 6
E_BLOCK = 2000

_mesh = plsc.VectorSubcoreMesh(core_axis_name="c", subcore_axis_name="s",
                               num_cores=1)
_sc_params = pltpu.CompilerParams(needs_layout_passes=False)


def _i32(x):
    return np.int32(x)


def _al(x):
    return pl.multiple_of(x, 8)


def _ssp(x):
    return jnp.maximum(x, 0.0) + jnp.log1p(jnp.exp(-jnp.abs(x))) - 0.6931471805599453


def _mlp_body(feat_ref, w1_ref, b1_ref, w2_ref, b2_ref, w3_ref, b3_ref, em_ref):
    h = jnp.dot(feat_ref[...], w1_ref[...], preferred_element_type=jnp.float32)
    h = _ssp(h + b1_ref[...])
    h = jnp.dot(h, w2_ref[...], preferred_element_type=jnp.float32)
    h = _ssp(h + b2_ref[...])
    em = jnp.dot(h, w3_ref[...], preferred_element_type=jnp.float32) + b3_ref[...]
    em_ref[...] = em


def _mlp_pallas(feat, W1, b1, W2, b2, W3, b3):
    grid = (E // E_BLOCK,)
    return pl.pallas_call(
        _mlp_body,
        grid=grid,
        in_specs=[
            pl.BlockSpec((E_BLOCK, 128), lambda i: (i, np.int32(0))),
            pl.BlockSpec((128, 128), lambda i: (np.int32(0), np.int32(0))),
            pl.BlockSpec((1, 128), lambda i: (np.int32(0), np.int32(0))),
            pl.BlockSpec((128, 64), lambda i: (np.int32(0), np.int32(0))),
            pl.BlockSpec((1, 64), lambda i: (np.int32(0), np.int32(0))),
            pl.BlockSpec((64, 1), lambda i: (np.int32(0), np.int32(0))),
            pl.BlockSpec((1, 1), lambda i: (np.int32(0), np.int32(0))),
        ],
        out_specs=pl.BlockSpec((E_BLOCK, 1), lambda i: (i, np.int32(0))),
        out_shape=jax.ShapeDtypeStruct((E, 1), jnp.float32),
    )(feat, W1, b1.reshape(1, 128), W2, b2.reshape(1, 64), W3, b3.reshape(1, 1))


def _digit_of(klo, khi, p):
    s = RBITS * p
    if s >= 32:
        return jnp.bitwise_and(
            lax.shift_right_logical(khi, _i32(s - 32)), _i32(R - 1))
    d = lax.shift_right_logical(klo, _i32(s))
    if s + RBITS > 32:
        nhi = s + RBITS - 32
        hi = lax.shift_left(
            jnp.bitwise_and(khi, _i32((1 << nhi) - 1)), _i32(32 - s))
        d = jnp.bitwise_or(jnp.bitwise_and(d, _i32((1 << (32 - s)) - 1)), hi)
    return jnp.bitwise_and(d, _i32(R - 1))


def _sort_body(klo_hbm, khi_hbm, ord_hbm,
               w_klo, w_khi, w_idx, w_pos,
               hist, tmp, acc, part, base,
               s_idx, s_grid):
    t = lax.axis_index("s").astype(jnp.int32)
    cbase = t * _i32(CHUNK)
    iota = lax.iota(jnp.int32, 16)
    z16 = jnp.zeros((16,), jnp.int32)

    def zero_r(ref):
        def zb(i, off):
            ref[pl.ds(off, 16)] = z16
            return off + _i32(16)
        lax.fori_loop(0, R // 16, zb, _i32(0), unroll=4)

    def stage(wb, p):
        if p == 0:
            pltpu.sync_copy(klo_hbm.at[pl.ds(_al(wb), W)], w_klo)
            pltpu.sync_copy(khi_hbm.at[pl.ds(_al(wb), W)], w_khi)
        else:
            pltpu.sync_copy(ord_hbm.at[pl.ds(_al(wb), W)], w_idx)
            pltpu.sync_copy(klo_hbm.at[w_idx], w_klo)
            pltpu.sync_copy(khi_hbm.at[w_idx], w_khi)

    for p in range(NPASS):
        zero_r(hist)

        def b1_win(w, wb, p=p):
            stage(wb, p)

            def vb(i, off, p=p):
                klo = w_klo[pl.ds(off, 16)]
                khi = w_khi[pl.ds(off, 16)]
                d = _digit_of(klo, khi, p)
                occ, last = plsc.scan_count(d)
                plsc.addupdate_scatter(hist, [d], occ, mask=last)
                return off + _i32(16)
            lax.fori_loop(0, W // 16, vb, _i32(0), unroll=4)
            return wb + _i32(W)
        lax.fori_loop(0, NWIN, b1_win, cbase)

        pltpu.sync_copy(hist, s_grid.at[t])
        plsc.subcore_barrier()

        zero_r(acc)
        zero_r(part)
        for tp in range(NT):
            pltpu.sync_copy(s_grid.at[_i32(tp)], tmp)
            fvec = z16 + (_i32(tp) < t).astype(jnp.int32)

            def ab(i, off):
                sl = pl.ds(off, 16)
                v = tmp[sl]
                acc[sl] = acc[sl] + v
                part[sl] = part[sl] + v * fvec
                return off + _i32(16)
            lax.fori_loop(0, R // 16, ab, _i32(0), unroll=4)

        def bb(i, carry):
            off, tot = carry
            sl = pl.ds(off, 16)
            a = acc[sl]
            incl = plsc.cumsum(a)
            base[sl] = (incl - a) + (z16 + tot) + part[sl]
            return (off + _i32(16), tot + jnp.sum(a, dtype=jnp.int32))
        lax.fori_loop(0, R // 16, bb, (_i32(0), _i32(0)))

        def b3_win(w, wb, p=p):
            stage(wb, p)

            def vb(i, off, p=p, wb=wb):
                sl = pl.ds(off, 16)
                klo = w_klo[sl]
                khi = w_khi[sl]
                if p == 0:
                    w_idx[sl] = (iota + off) + wb
                d = _digit_of(klo, khi, p)
                occ, last = plsc.scan_count(d)
                b = plsc.load_gather(base, [d])
                w_pos[sl] = b + occ - _i32(1)
                plsc.addupdate_scatter(base, [d], occ, mask=last)
                return off + _i32(16)
            lax.fori_loop(0, W // 16, vb, _i32(0))
            pltpu.sync_copy(w_idx, s_idx.at[w_pos])
            return wb + _i32(W)
        lax.fori_loop(0, NWIN, b3_win, cbase)

        plsc.subcore_barrier()

        def b4(w, wb):
            pltpu.sync_copy(s_idx.at[pl.ds(_al(wb), W)], w_idx)
            pltpu.sync_copy(w_idx, ord_hbm.at[pl.ds(_al(wb), W)])
            return wb + _i32(W)
        lax.fori_loop(0, NWIN, b4, cbase)
        plsc.subcore_barrier()


def _sort_kernel(klo, khi):
    f = pl.kernel(
        _sort_body,
        out_type=jax.ShapeDtypeStruct((N2,), jnp.int32),
        mesh=_mesh,
        compiler_params=_sc_params,
        scratch_types=[
            pltpu.VMEM((W,), jnp.int32),
            pltpu.VMEM((W,), jnp.int32),
            pltpu.VMEM((W,), jnp.int32),
            pltpu.VMEM((W,), jnp.int32),
            pltpu.VMEM((R,), jnp.int32),
            pltpu.VMEM((R,), jnp.int32),
            pltpu.VMEM((R,), jnp.int32),
            pltpu.VMEM((R,), jnp.int32),
            pltpu.VMEM((R,), jnp.int32),
            pltpu.VMEM_SHARED((N2,), jnp.int32),
            pltpu.VMEM_SHARED((NT, R), jnp.int32),
        ],
    )
    return f(klo, khi)


def _pair_body(em_hbm, ec_hbm, ord_hbm, real2_hbm, ec2_hbm,
               ux_hbm, uy_hbm, uz_hbm,
               forces_hbm, emc_hbm,
               w_em, w_j, w_e, w_real, w_ec2,
               w_ux, w_uy, w_uz, w_fx, w_fy, w_fz,
               s_sums, s_cnt, s_forces):
    t = lax.axis_index("s").astype(jnp.int32)
    iota = lax.iota(jnp.int32, 16)
    onef = jnp.full((16,), 1.0, jnp.float32)
    zf16 = jnp.zeros((16,), jnp.float32)

    def of(i, off):
        w_fz[pl.ds(off, 16)] = onef
        w_fx[pl.ds(off, 16)] = zf16
        return off + _i32(16)
    lax.fori_loop(0, W // 16, of, _i32(0), unroll=4)

    nwords = FP * 3 // NT
    rb = t * _i32(nwords)
    pltpu.sync_copy(w_fx.at[pl.ds(0, nwords)], s_forces.at[pl.ds(_al(rb), nwords)])
    nsum = FP // NT
    sb = t * _i32(nsum)
    pltpu.sync_copy(w_fx.at[pl.ds(0, nsum)], s_sums.at[pl.ds(_al(sb), nsum)])
    pltpu.sync_copy(w_fx.at[pl.ds(0, nsum)], s_cnt.at[pl.ds(_al(sb), nsum)])
    plsc.subcore_barrier()

    ebase = t * _i32(EC)

    def a1(w, wb):
        pltpu.sync_copy(em_hbm.at[pl.ds(_al(wb), W)], w_em)
        pltpu.sync_copy(ec_hbm.at[pl.ds(_al(wb), W)], w_j)
        pltpu.sync_copy(w_em, s_sums.at[w_j], add=True)
        pltpu.sync_copy(w_fz, s_cnt.at[w_j], add=True)
        return wb + _i32(W)
    lax.fori_loop(0, NWE, a1, ebase)
    plsc.subcore_barrier()

    def a2(w, wb):
        pltpu.sync_copy(em_hbm.at[pl.ds(_al(wb), W)], w_em)
        pltpu.sync_copy(ec_hbm.at[pl.ds(_al(wb), W)], w_j)
        pltpu.sync_copy(s_sums.at[w_j], w_ux)
        pltpu.sync_copy(s_cnt.at[w_j], w_uy)

        def vb(i, off):
            sl = pl.ds(off, 16)
            w_em[sl] = w_em[sl] - w_ux[sl] / jnp.maximum(w_uy[sl], 1.0)
            return off + _i32(16)
        lax.fori_loop(0, W // 16, vb, _i32(0), unroll=4)
        pltpu.sync_copy(w_em, emc_hbm.at[pl.ds(_al(wb), W)])
        return wb + _i32(W)
    lax.fori_loop(0, NWE, a2, ebase)
    plsc.subcore_barrier()

    def c_win(w, wb):
        pltpu.sync_copy(ord_hbm.at[pl.ds(_al(wb), W)], w_j)

        def v1(i, off):
            sl = pl.ds(off, 16)
            j = w_j[sl]
            ge = (j >= _i32(E)).astype(jnp.int32)
            w_e[sl] = j - ge * _i32(E)
            return off + _i32(16)
        lax.fori_loop(0, W // 16, v1, _i32(0), unroll=4)

        pltpu.sync_copy(emc_hbm.at[w_e], w_em)
        pltpu.sync_copy(real2_hbm.at[w_j], w_real)
        pltpu.sync_copy(ec2_hbm.at[w_j], w_ec2)
        pltpu.sync_copy(ux_hbm.at[w_j], w_ux)
        pltpu.sync_copy(uy_hbm.at[w_j], w_uy)
        pltpu.sync_copy(uz_hbm.at[w_j], w_uz)

        def v3(i, off):
            sl = pl.ds(off, 16)
            rows = iota + off
            prow = jnp.bitwise_xor(rows, _i32(1))
            rl = w_real[sl]
            prl = plsc.load_gather(w_real, [prow])
            emv = w_em[sl] * rl
            pem = plsc.load_gather(w_em, [prow]) * prl
            valid = (rl * prl) > 0.5
            ne = jnp.where(valid, (emv + pem) * 0.5, emv)
            e3 = w_ec2[sl] * _i32(3)
            w_fx[sl] = ne * w_ux[sl]
            w_fy[sl] = ne * w_uy[sl]
            w_fz[sl] = ne * w_uz[sl]
            w_e[sl] = e3
            w_j[sl] = e3 + _i32(1)
            w_ec2[sl] = e3 + _i32(2)
            return off + _i32(16)
        lax.fori_loop(0, W // 16, v3, _i32(0), unroll=4)

        pltpu.sync_copy(w_fx, s_forces.at[w_e], add=True)
        pltpu.sync_copy(w_fy, s_forces.at[w_j], add=True)
        pltpu.sync_copy(w_fz, s_forces.at[w_ec2], add=True)

        return wb + _i32(W)
    lax.fori_loop(0, NWIN, c_win, t * _i32(CHUNK))
    plsc.subcore_barrier()

    pltpu.sync_copy(s_forces.at[pl.ds(_al(rb), nwords)],
                    w_fx.at[pl.ds(0, nwords)])
    pltpu.sync_copy(w_fx.at[pl.ds(0, nwords)],
                    forces_hbm.at[pl.ds(_al(rb), nwords)])


def _pair_kernel(em, ec, rec, real2, ec2, ux, uy, uz):
    f = pl.kernel(
        _pair_body,
        out_type=(jax.ShapeDtypeStruct((FP * 3,), jnp.float32),
                  jax.ShapeDtypeStruct((E,), jnp.float32)),
        mesh=_mesh,
        compiler_params=_sc_params,
        scratch_types=[
            pltpu.VMEM((W,), jnp.float32),
            pltpu.VMEM((W,), jnp.int32),
            pltpu.VMEM((W,), jnp.int32),
            pltpu.VMEM((W,), jnp.float32),
            pltpu.VMEM((W,), jnp.int32),
            pltpu.VMEM((W,), jnp.float32),
            pltpu.VMEM((W,), jnp.float32),
            pltpu.VMEM((W,), jnp.float32),
            pltpu.VMEM((W,), jnp.float32),
            pltpu.VMEM((W,), jnp.float32),
            pltpu.VMEM((W,), jnp.float32),
            pltpu.VMEM_SHARED((FP,), jnp.float32),
            pltpu.VMEM_SHARED((FP,), jnp.float32),
            pltpu.VMEM_SHARED((FP * 3,), jnp.float32),
        ],
    )
    return f(em, ec, rec, real2, ec2, ux, uy, uz)


def kernel(features_for_direct_force, edge_vectors, edge_lengths, pos, edge_index,
           W1, b1, W2, b2, W3, b3):
    ec64 = edge_index[0]
    en64 = edge_index[1]
    ec = ec64.astype(jnp.int32)
    en = en64.astype(jnp.int32)
    unit_vec = edge_vectors / edge_lengths[:, None]

    em = _mlp_pallas(features_for_direct_force, W1, b1, W2, b2, W3, b3)[:, 0]

    gm = en64 > ec64[-1]
    gmf = gm.astype(jnp.float32)
    real2 = jnp.concatenate([jnp.ones((E,), jnp.float32), gmf])
    zero_e = jnp.zeros((E,), jnp.float32)
    ux = jnp.concatenate([unit_vec[:, 0], jnp.where(gm, -unit_vec[:, 0], zero_e)])
    uy = jnp.concatenate([unit_vec[:, 1], jnp.where(gm, -unit_vec[:, 1], zero_e)])
    uz = jnp.concatenate([unit_vec[:, 2], jnp.where(gm, -unit_vec[:, 2], zero_e)])
    ec2 = jnp.concatenate([ec, jnp.where(gm, en, jnp.int32(NNODES))])
    en2 = jnp.concatenate([en, jnp.where(gm, ec, jnp.int32(0))])
    el2 = jnp.concatenate([edge_lengths, jnp.where(gm, edge_lengths, zero_e)])
    s_abs = jnp.abs(unit_vec).sum(axis=1)
    uvs2 = jnp.concatenate([s_abs, jnp.where(gm, s_abs, zero_e)])

    key = (ec2.astype(jnp.int64) + en2.astype(jnp.int64)
           + (1e10 * el2).astype(jnp.int64)
           + (1e10 * uvs2).astype(jnp.int64))
    is_real = jnp.concatenate([jnp.ones((E,), bool), gm])
    key = jnp.where(is_real, key, jnp.iinfo(jnp.int64).max)
    klo = (key & 0xFFFFFFFF).astype(jnp.int32)
    khi = lax.shift_right_logical(key, np.int64(32)).astype(jnp.int32)

    rec = _sort_kernel(klo, khi)

    forces_fp, _emc = _pair_kernel(em, ec, rec, real2, ec2, ux, uy, uz)
    return forces_fp.reshape(FP, 3)[:NNODES]


# speedup vs baseline: 4.0772x; 1.1920x over previous
import functools

import numpy as np
import jax
import jax.numpy as jnp
from jax import lax
from jax.experimental import pallas as pl
from jax.experimental.pallas import tpu as pltpu
from jax.experimental.pallas import tpu_sc as plsc

E = 320000
N2 = 2 * E
NNODES = 10000
FP = 10240
NT = 16
CHUNK = N2 // NT
W = 10000
NWIN = CHUNK // W
EC = E // NT
NWE = EC // W
WS = 8000
NWINS = CHUNK // WS
RBITS = 11
R = 1 << RBITS
NPASS = 6
E_BLOCK = 2000

_mesh = plsc.VectorSubcoreMesh(core_axis_name="c", subcore_axis_name="s",
                               num_cores=1)
_sc_params = pltpu.CompilerParams(needs_layout_passes=False)


def _i32(x):
    return np.int32(x)


def _al(x):
    return pl.multiple_of(x, 8)


def _ssp(x):
    return jnp.maximum(x, 0.0) + jnp.log1p(jnp.exp(-jnp.abs(x))) - 0.6931471805599453


def _mlp_body(feat_ref, w1_ref, b1_ref, w2_ref, b2_ref, w3_ref, b3_ref, em_ref):
    h = jnp.dot(feat_ref[...], w1_ref[...], preferred_element_type=jnp.float32)
    h = _ssp(h + b1_ref[...])
    h = jnp.dot(h, w2_ref[...], preferred_element_type=jnp.float32)
    h = _ssp(h + b2_ref[...])
    em = jnp.dot(h, w3_ref[...], preferred_element_type=jnp.float32) + b3_ref[...]
    em_ref[...] = em


def _mlp_pallas(feat, W1, b1, W2, b2, W3, b3):
    grid = (E // E_BLOCK,)
    return pl.pallas_call(
        _mlp_body,
        grid=grid,
        in_specs=[
            pl.BlockSpec((E_BLOCK, 128), lambda i: (i, np.int32(0))),
            pl.BlockSpec((128, 128), lambda i: (np.int32(0), np.int32(0))),
            pl.BlockSpec((1, 128), lambda i: (np.int32(0), np.int32(0))),
            pl.BlockSpec((128, 64), lambda i: (np.int32(0), np.int32(0))),
            pl.BlockSpec((1, 64), lambda i: (np.int32(0), np.int32(0))),
            pl.BlockSpec((64, 1), lambda i: (np.int32(0), np.int32(0))),
            pl.BlockSpec((1, 1), lambda i: (np.int32(0), np.int32(0))),
        ],
        out_specs=pl.BlockSpec((E_BLOCK, 1), lambda i: (i, np.int32(0))),
        out_shape=jax.ShapeDtypeStruct((E, 1), jnp.float32),
    )(feat, W1, b1.reshape(1, 128), W2, b2.reshape(1, 64), W3, b3.reshape(1, 1))


def _digit_of(klo, khi, p):
    s = RBITS * p
    if s >= 32:
        return jnp.bitwise_and(
            lax.shift_right_logical(khi, _i32(s - 32)), _i32(R - 1))
    d = lax.shift_right_logical(klo, _i32(s))
    if s + RBITS > 32:
        nhi = s + RBITS - 32
        hi = lax.shift_left(
            jnp.bitwise_and(khi, _i32((1 << nhi) - 1)), _i32(32 - s))
        d = jnp.bitwise_or(jnp.bitwise_and(d, _i32((1 << (32 - s)) - 1)), hi)
    return jnp.bitwise_and(d, _i32(R - 1))


def _sort_body(klo_hbm, khi_hbm, ord_hbm,
               w_klo, w_khi, w_idx, w_pos, dcache,
               hist, tmp, acc, part, base,
               s_idx, s_grid):
    t = lax.axis_index("s").astype(jnp.int32)
    cbase = t * _i32(CHUNK)
    iota = lax.iota(jnp.int32, 16)
    z16 = jnp.zeros((16,), jnp.int32)

    def zero_r(ref):
        def zb(i, off):
            ref[pl.ds(off, 16)] = z16
            return off + _i32(16)
        lax.fori_loop(0, R // 16, zb, _i32(0), unroll=4)

    def stage(wb, p):
        if p == 0:
            pltpu.sync_copy(klo_hbm.at[pl.ds(_al(wb), WS)], w_klo)
            pltpu.sync_copy(khi_hbm.at[pl.ds(_al(wb), WS)], w_khi)
        else:
            pltpu.sync_copy(ord_hbm.at[pl.ds(_al(wb), WS)], w_idx)
            pltpu.sync_copy(klo_hbm.at[w_idx], w_klo)
            pltpu.sync_copy(khi_hbm.at[w_idx], w_khi)

    for p in range(NPASS):
        zero_r(hist)

        def b1_win(w, wb, p=p):
            stage(wb, p)
            loc = wb - cbase

            def vb(i, carry, p=p):
                ow, oc = carry
                klo = w_klo[pl.ds(ow, 16)]
                khi = w_khi[pl.ds(ow, 16)]
                d = _digit_of(klo, khi, p)
                dcache[pl.ds(oc, 16)] = d
                occ, last = plsc.scan_count(d)
                plsc.addupdate_scatter(hist, [d], occ, mask=last)
                return (ow + _i32(16), oc + _i32(16))
            lax.fori_loop(0, WS // 16, vb, (_i32(0), loc), unroll=4)
            return wb + _i32(WS)
        lax.fori_loop(0, NWINS, b1_win, cbase)

        pltpu.sync_copy(hist, s_grid.at[t])
        plsc.subcore_barrier()

        zero_r(acc)
        zero_r(part)
        for tp in range(NT):
            pltpu.sync_copy(s_grid.at[_i32(tp)], tmp)
            fvec = z16 + (_i32(tp) < t).astype(jnp.int32)

            def ab(i, off):
                sl = pl.ds(off, 16)
                v = tmp[sl]
                acc[sl] = acc[sl] + v
                part[sl] = part[sl] + v * fvec
                return off + _i32(16)
            lax.fori_loop(0, R // 16, ab, _i32(0), unroll=4)

        def bb(i, carry):
            off, tot = carry
            sl = pl.ds(off, 16)
            a = acc[sl]
            incl = plsc.cumsum(a)
            base[sl] = (incl - a) + (z16 + tot) + part[sl]
            return (off + _i32(16), tot + jnp.sum(a, dtype=jnp.int32))
        lax.fori_loop(0, R // 16, bb, (_i32(0), _i32(0)))

        def b3_win(w, wb, p=p):
            if p > 0:
                pltpu.sync_copy(ord_hbm.at[pl.ds(_al(wb), WS)], w_idx)
            loc = wb - cbase

            def vb(i, carry, p=p, wb=wb):
                ow, oc = carry
                sl = pl.ds(ow, 16)
                if p == 0:
                    w_idx[sl] = (iota + ow) + wb
                d = dcache[pl.ds(oc, 16)]
                occ, last = plsc.scan_count(d)
                b = plsc.load_gather(base, [d])
                w_pos[sl] = b + occ - _i32(1)
                plsc.addupdate_scatter(base, [d], occ, mask=last)
                return (ow + _i32(16), oc + _i32(16))
            lax.fori_loop(0, WS // 16, vb, (_i32(0), loc))
            pltpu.sync_copy(w_idx, s_idx.at[w_pos])
            return wb + _i32(WS)
        lax.fori_loop(0, NWINS, b3_win, cbase)

        plsc.subcore_barrier()

        def b4(w, wb):
            pltpu.sync_copy(s_idx.at[pl.ds(_al(wb), WS)], w_idx)
            pltpu.sync_copy(w_idx, ord_hbm.at[pl.ds(_al(wb), WS)])
            return wb + _i32(WS)
        lax.fori_loop(0, NWINS, b4, cbase)
        plsc.subcore_barrier()


def _sort_kernel(klo, khi):
    f = pl.kernel(
        _sort_body,
        out_type=jax.ShapeDtypeStruct((N2,), jnp.int32),
        mesh=_mesh,
        compiler_params=_sc_params,
        scratch_types=[
            pltpu.VMEM((WS,), jnp.int32),
            pltpu.VMEM((WS,), jnp.int32),
            pltpu.VMEM((WS,), jnp.int32),
            pltpu.VMEM((WS,), jnp.int32),
            pltpu.VMEM((CHUNK,), jnp.int32),
            pltpu.VMEM((R,), jnp.int32),
            pltpu.VMEM((R,), jnp.int32),
            pltpu.VMEM((R,), jnp.int32),
            pltpu.VMEM((R,), jnp.int32),
            pltpu.VMEM((R,), jnp.int32),
            pltpu.VMEM_SHARED((N2,), jnp.int32),
            pltpu.VMEM_SHARED((NT, R), jnp.int32),
        ],
    )
    return f(klo, khi)


def _pair_body(em_hbm, ec_hbm, ord_hbm, real2_hbm, ec2_hbm,
               ux_hbm, uy_hbm, uz_hbm,
               forces_hbm, emc_hbm,
               w_em, w_j, w_e, w_real, w_ec2,
               w_ux, w_uy, w_uz, w_fx, w_fy, w_fz,
               s_sums, s_cnt, s_forces):
    t = lax.axis_index("s").astype(jnp.int32)
    iota = lax.iota(jnp.int32, 16)
    onef = jnp.full((16,), 1.0, jnp.float32)
    zf16 = jnp.zeros((16,), jnp.float32)

    def of(i, off):
        w_fz[pl.ds(off, 16)] = onef
        w_fx[pl.ds(off, 16)] = zf16
        return off + _i32(16)
    lax.fori_loop(0, W // 16, of, _i32(0), unroll=4)

    nwords = FP * 3 // NT
    rb = t * _i32(nwords)
    pltpu.sync_copy(w_fx.at[pl.ds(0, nwords)], s_forces.at[pl.ds(_al(rb), nwords)])
    nsum = FP // NT
    sb = t * _i32(nsum)
    pltpu.sync_copy(w_fx.at[pl.ds(0, nsum)], s_sums.at[pl.ds(_al(sb), nsum)])
    pltpu.sync_copy(w_fx.at[pl.ds(0, nsum)], s_cnt.at[pl.ds(_al(sb), nsum)])
    plsc.subcore_barrier()

    ebase = t * _i32(EC)

    def a1(w, wb):
        pltpu.sync_copy(em_hbm.at[pl.ds(_al(wb), W)], w_em)
        pltpu.sync_copy(ec_hbm.at[pl.ds(_al(wb), W)], w_j)
        pltpu.sync_copy(w_em, s_sums.at[w_j], add=True)
        pltpu.sync_copy(w_fz, s_cnt.at[w_j], add=True)
        return wb + _i32(W)
    lax.fori_loop(0, NWE, a1, ebase)
    plsc.subcore_barrier()

    def a2(w, wb):
        pltpu.sync_copy(em_hbm.at[pl.ds(_al(wb), W)], w_em)
        pltpu.sync_copy(ec_hbm.at[pl.ds(_al(wb), W)], w_j)
        pltpu.sync_copy(s_sums.at[w_j], w_ux)
        pltpu.sync_copy(s_cnt.at[w_j], w_uy)

        def vb(i, off):
            sl = pl.ds(off, 16)
            w_em[sl] = w_em[sl] - w_ux[sl] / jnp.maximum(w_uy[sl], 1.0)
            return off + _i32(16)
        lax.fori_loop(0, W // 16, vb, _i32(0), unroll=4)
        pltpu.sync_copy(w_em, emc_hbm.at[pl.ds(_al(wb), W)])
        return wb + _i32(W)
    lax.fori_loop(0, NWE, a2, ebase)
    plsc.subcore_barrier()

    def c_win(w, wb):
        pltpu.sync_copy(ord_hbm.at[pl.ds(_al(wb), W)], w_j)

        def v1(i, off):
            sl = pl.ds(off, 16)
            j = w_j[sl]
            ge = (j >= _i32(E)).astype(jnp.int32)
            w_e[sl] = j - ge * _i32(E)
            return off + _i32(16)
        lax.fori_loop(0, W // 16, v1, _i32(0), unroll=4)

        pltpu.sync_copy(emc_hbm.at[w_e], w_em)
        pltpu.sync_copy(real2_hbm.at[w_j], w_real)
        pltpu.sync_copy(ec2_hbm.at[w_j], w_ec2)
        pltpu.sync_copy(ux_hbm.at[w_j], w_ux)
        pltpu.sync_copy(uy_hbm.at[w_j], w_uy)
        pltpu.sync_copy(uz_hbm.at[w_j], w_uz)

        def v3(i, off):
            sl = pl.ds(off, 16)
            rows = iota + off
            prow = jnp.bitwise_xor(rows, _i32(1))
            rl = w_real[sl]
            prl = plsc.load_gather(w_real, [prow])
            emv = w_em[sl] * rl
            pem = plsc.load_gather(w_em, [prow]) * prl
            valid = (rl * prl) > 0.5
            ne = jnp.where(valid, (emv + pem) * 0.5, emv)
            e3 = w_ec2[sl] * _i32(3)
            w_fx[sl] = ne * w_ux[sl]
            w_fy[sl] = ne * w_uy[sl]
            w_fz[sl] = ne * w_uz[sl]
            w_e[sl] = e3
            w_j[sl] = e3 + _i32(1)
            w_ec2[sl] = e3 + _i32(2)
            return off + _i32(16)
        lax.fori_loop(0, W // 16, v3, _i32(0), unroll=4)

        pltpu.sync_copy(w_fx, s_forces.at[w_e], add=True)
        pltpu.sync_copy(w_fy, s_forces.at[w_j], add=True)
        pltpu.sync_copy(w_fz, s_forces.at[w_ec2], add=True)

        return wb + _i32(W)
    lax.fori_loop(0, NWIN, c_win, t * _i32(CHUNK))
    plsc.subcore_barrier()

    pltpu.sync_copy(s_forces.at[pl.ds(_al(rb), nwords)],
                    w_fx.at[pl.ds(0, nwords)])
    pltpu.sync_copy(w_fx.at[pl.ds(0, nwords)],
                    forces_hbm.at[pl.ds(_al(rb), nwords)])


def _pair_kernel(em, ec, rec, real2, ec2, ux, uy, uz):
    f = pl.kernel(
        _pair_body,
        out_type=(jax.ShapeDtypeStruct((FP * 3,), jnp.float32),
                  jax.ShapeDtypeStruct((E,), jnp.float32)),
        mesh=_mesh,
        compiler_params=_sc_params,
        scratch_types=[
            pltpu.VMEM((W,), jnp.float32),
            pltpu.VMEM((W,), jnp.int32),
            pltpu.VMEM((W,), jnp.int32),
            pltpu.VMEM((W,), jnp.float32),
            pltpu.VMEM((W,), jnp.int32),
            pltpu.VMEM((W,), jnp.float32),
            pltpu.VMEM((W,), jnp.float32),
            pltpu.VMEM((W,), jnp.float32),
            pltpu.VMEM((W,), jnp.float32),
            pltpu.VMEM((W,), jnp.float32),
            pltpu.VMEM((W,), jnp.float32),
            pltpu.VMEM_SHARED((FP,), jnp.float32),
            pltpu.VMEM_SHARED((FP,), jnp.float32),
            pltpu.VMEM_SHARED((FP * 3,), jnp.float32),
        ],
    )
    return f(em, ec, rec, real2, ec2, ux, uy, uz)


def kernel(features_for_direct_force, edge_vectors, edge_lengths, pos, edge_index,
           W1, b1, W2, b2, W3, b3):
    ec64 = edge_index[0]
    en64 = edge_index[1]
    ec = ec64.astype(jnp.int32)
    en = en64.astype(jnp.int32)
    unit_vec = edge_vectors / edge_lengths[:, None]

    em = _mlp_pallas(features_for_direct_force, W1, b1, W2, b2, W3, b3)[:, 0]

    gm = en64 > ec64[-1]
    gmf = gm.astype(jnp.float32)
    real2 = jnp.concatenate([jnp.ones((E,), jnp.float32), gmf])
    zero_e = jnp.zeros((E,), jnp.float32)
    ux = jnp.concatenate([unit_vec[:, 0], jnp.where(gm, -unit_vec[:, 0], zero_e)])
    uy = jnp.concatenate([unit_vec[:, 1], jnp.where(gm, -unit_vec[:, 1], zero_e)])
    uz = jnp.concatenate([unit_vec[:, 2], jnp.where(gm, -unit_vec[:, 2], zero_e)])
    ec2 = jnp.concatenate([ec, jnp.where(gm, en, jnp.int32(NNODES))])
    en2 = jnp.concatenate([en, jnp.where(gm, ec, jnp.int32(0))])
    el2 = jnp.concatenate([edge_lengths, jnp.where(gm, edge_lengths, zero_e)])
    s_abs = jnp.abs(unit_vec).sum(axis=1)
    uvs2 = jnp.concatenate([s_abs, jnp.where(gm, s_abs, zero_e)])

    key = (ec2.astype(jnp.int64) + en2.astype(jnp.int64)
           + (1e10 * el2).astype(jnp.int64)
           + (1e10 * uvs2).astype(jnp.int64))
    is_real = jnp.concatenate([jnp.ones((E,), bool), gm])
    key = jnp.where(is_real, key, jnp.iinfo(jnp.int64).max)
    klo = (key & 0xFFFFFFFF).astype(jnp.int32)
    khi = lax.shift_right_logical(key, np.int64(32)).astype(jnp.int32)

    rec = _sort_kernel(klo, khi)

    forces_fp, _emc = _pair_kernel(em, ec, rec, real2, ec2, ux, uy, uz)
    return forces_fp.reshape(FP, 3)[:NNODES]
